# baseline (speedup 1.0000x reference)
import functools

import jax
import jax.numpy as jnp
from jax import lax
from jax.experimental import pallas as pl
from jax.experimental.pallas import tpu as pltpu
from jax.experimental.pallas import tpu_sc as plsc

_B = 4096
_N = 100
_NP = 104
_NNODES = 100000
_NEDGES = 1600000
_RAW = 128
_ERAW = 16
_TD = 8
_D = 64
_DA = 128

_R = _B * _NP

_NC = 2
_NS = 16
_NW = _NC * _NS
_PW = _R // _NW
_CH = 416
_NCHUNK = _PW // _CH
_BD = _B // _NW

_BBLK = 64
_RBLK = _BBLK * _NP

_NS16 = 16
_BP = 512


def _proj_body(x_ref, w_ref, b_ref, o_ref):
    x = x_ref[...]
    proj = (jnp.dot(x, w_ref[...], preferred_element_type=jnp.float32)
            + b_ref[...])
    rb = x.shape[0]
    o_ref[...] = jnp.concatenate(
        [proj, x[:, 0:1], jnp.zeros((rb, _DA - _D - 1), jnp.float32)],
        axis=1)


def _project_table(node_raw, w_feat, b_feat):
    rb = 2000
    return pl.pallas_call(
        _proj_body,
        grid=(_NNODES // rb,),
        in_specs=[
            pl.BlockSpec((rb, _RAW), lambda i: (i, 0)),
            pl.BlockSpec((_RAW, _D), lambda i: (0, 0)),
            pl.BlockSpec((1, _D), lambda i: (0, 0)),
        ],
        out_specs=pl.BlockSpec((rb, _DA), lambda i: (i, 0)),
        out_shape=jax.ShapeDtypeStruct((_NNODES, _DA), jnp.float32),
    )(node_raw, w_feat, b_feat.reshape(1, _D))


def _sc_gather(p_tab, edge_col0, ids_flat, eids_flat, dst):
    mesh = plsc.VectorSubcoreMesh(
        core_axis_name="c", subcore_axis_name="s",
        num_cores=_NC, num_subcores=_NS,
    )
    out_type = [
        jax.ShapeDtypeStruct((_R, _DA), jnp.float32),
        jax.ShapeDtypeStruct((_R,), jnp.float32),
        jax.ShapeDtypeStruct((_B, _DA), jnp.float32),
    ]
    chunk_bufs = lambda: [
        pltpu.VMEM((_CH,), jnp.int32),
        pltpu.VMEM((_CH,), jnp.int32),
        pltpu.VMEM((_CH, _DA), jnp.float32),
        pltpu.VMEM((_CH,), jnp.float32),
        pltpu.SemaphoreType.DMA,
        pltpu.SemaphoreType.DMA,
        pltpu.SemaphoreType.DMA,
    ]
    scratch = chunk_bufs() + chunk_bufs() + [
        pltpu.VMEM((_BD,), jnp.int32),
    ]

    @functools.partial(pl.kernel, mesh=mesh, out_type=out_type,
                       scratch_types=scratch,
                       compiler_params=pltpu.CompilerParams(
                           use_tc_tiling_on_sc=False,
                           needs_layout_passes=False))
    def k(p_h, e0t_h, ids_h, eids_h, dst_h,
          nf_o, e0_o, pd_o,
          idx_a, eidx_a, rows_a, e0_a, sga, sea, swa,
          idx_b, eidx_b, rows_b, e0_b, sgb, seb, swb,
          didx_v):
        wid = lax.axis_index("s") * _NC + lax.axis_index("c")
        base = wid * _PW

        bufs_a = (idx_a, eidx_a, rows_a, e0_a, sga, sea, swa)
        bufs_b = (idx_b, eidx_b, rows_b, e0_b, sgb, seb, swb)

        def load_idx(bufs, off):
            pltpu.sync_copy(ids_h.at[pl.ds(off, _CH)], bufs[0])
            pltpu.sync_copy(eids_h.at[pl.ds(off, _CH)], bufs[1])

        def issue_gathers(bufs):
            idx, eidx, rows, e0_v, sg, se = bufs[:6]
            return (pltpu.async_copy(p_h.at[idx], rows, sg),
                    pltpu.async_copy(e0t_h.at[eidx], e0_v, se))

        def issue_writebacks(bufs, off):
            rows, e0_v, sw = bufs[2], bufs[3], bufs[6]
            return (pltpu.async_copy(rows, nf_o.at[pl.ds(off, _CH)], sw),
                    pltpu.async_copy(e0_v, e0_o.at[pl.ds(off, _CH)], sw))

        def pair(ci, carry):
            off_a = base + (2 * ci) * _CH
            off_b = off_a + _CH
            load_idx(bufs_a, off_a)
            ga = issue_gathers(bufs_a)
            load_idx(bufs_b, off_b)
            gb = issue_gathers(bufs_b)
            for cp in ga:
                cp.wait()
            wa = issue_writebacks(bufs_a, off_a)
            for cp in gb:
                cp.wait()
            wb = issue_writebacks(bufs_b, off_b)
            for cp in wa + wb:
                cp.wait()
            return carry

        lax.fori_loop(0, _NCHUNK // 2, pair, 0)

        dbase = wid * _BD
        pltpu.sync_copy(dst_h.at[pl.ds(dbase, _BD)], didx_v)
        drows = rows_a.at[pl.ds(0, _BD)]
        pltpu.async_copy(p_h.at[didx_v], drows, sga).wait()
        pltpu.sync_copy(drows, pd_o.at[pl.ds(dbase, _BD)])

    return k(p_tab, edge_col0, ids_flat, eids_flat, dst)


def _pre_body(ids_ref, t_ref, e0_ref, dst_ref, twb_ref, o_ref):
    ids = ids_ref[...]
    t = t_ref[...]
    e0w = e0_ref[...]
    dstc = dst_ref[...]

    valid = (ids > 0).astype(jnp.int32)
    vi = valid.sum(axis=-1, keepdims=True)
    n_l = lax.broadcasted_iota(jnp.int32, (_BP, _NP), 1)
    has_in = ((n_l >= 1) & (n_l <= vi - 1)).astype(jnp.float32)
    hprev = ((n_l - 1 >= 1) & (n_l - 1 <= vi - 1)).astype(jnp.float32)
    inv_deg = 1.0 - 0.5 * has_in
    coef = has_in * lax.rsqrt((1.0 + hprev) * (1.0 + has_in))
    co = (ids == dstc).astype(jnp.float32)

    planes = [e0w, co]
    for k in range(_TD):
        planes.append(jnp.cos(t * twb_ref[0, k] + twb_ref[1, k]))
    planes.append(inv_deg)
    planes.append(coef)
    z = jnp.zeros_like(e0w)
    planes.extend([z, z, z, z])
    o_ref[...] = jnp.stack(planes, axis=0)


def _pre(ids2, times2, e02, dstc, tw_tb):
    return pl.pallas_call(
        _pre_body,
        grid=(_B // _BP,),
        in_specs=[
            pl.BlockSpec((_BP, _NP), lambda i: (i, 0)),
            pl.BlockSpec((_BP, _NP), lambda i: (i, 0)),
            pl.BlockSpec((_BP, _NP), lambda i: (i, 0)),
            pl.BlockSpec((_BP, 1), lambda i: (i, 0)),
            pl.BlockSpec(memory_space=pltpu.SMEM),
        ],
        out_specs=pl.BlockSpec((_NS16, _BP, _NP), lambda i: (0, i, 0)),
        out_shape=jax.ShapeDtypeStruct((_NS16, _B, _NP), jnp.float32),
    )(ids2, times2, e02, dstc, tw_tb)


def _fuse_body(nf_ref, s_ref, pd_ref, wtil_ref, ws_ref, bs_ref,
               lg_ref, lb_ref, wg_ref, bg_ref, wo_ref, bo_ref,
               src_ref, dstout_ref):
    x = nf_ref[...]
    s = s_ref[...]
    pd = pd_ref[...]

    rsk = x[:, _D:_D + 1]
    cskb = pd[:, _D:_D + 1]
    csk = jnp.broadcast_to(cskb.reshape(_BBLK, 1, 1),
                           (_BBLK, _NP, 1)).reshape(_RBLK, 1)
    ss = (rsk.astype(jnp.int32) == csk.astype(jnp.int32)).astype(jnp.float32)

    acc = (x + jnp.dot(s, wtil_ref[...], preferred_element_type=jnp.float32)
           + ss * ws_ref[...] + bs_ref[...])

    lane = lax.broadcasted_iota(jnp.int32, (1, _DA), 1)
    lm = (lane < _D).astype(jnp.float32)
    am = acc * lm
    mu = am.sum(axis=-1, keepdims=True) * (1.0 / _D)
    xc = (acc - mu) * lm
    var = (xc * xc).sum(axis=-1, keepdims=True) * (1.0 / _D)
    fused = xc * lax.rsqrt(var + 1e-5) * lg_ref[...] + lb_ref[...]

    inv_deg = s[:, 10:11]
    coef = s[:, 11:12]
    xprev = pltpu.roll(fused, 1, 0)
    agg = fused * inv_deg + coef * xprev

    gcn = jnp.maximum(
        jnp.dot(agg, wg_ref[...], preferred_element_type=jnp.float32)
        + bg_ref[...], 0.0)
    y = jnp.dot(gcn, wo_ref[:_D, :], preferred_element_type=jnp.float32)
    pooled = y.reshape(_BBLK, _NP, _D)[:, :_N, :].sum(axis=1) * (1.0 / _N)
    src_ref[...] = pooled + bo_ref[...]
    dstout_ref[...] = (
        jnp.dot(pd, wo_ref[...], preferred_element_type=jnp.float32)
        + bo_ref[...])


def _fuse(nf, s16, pdst, wtil, w_struct, bias_sum, ln_g, ln_b,
          wg_aug, b_gcn, wo_aug, b_out):
    grid = (_B // _BBLK,)
    wspec = lambda shape: pl.BlockSpec(shape, lambda i: (0,) * len(shape))
    return pl.pallas_call(
        _fuse_body,
        grid=grid,
        in_specs=[
            pl.BlockSpec((_RBLK, _DA), lambda i: (i, 0)),
            pl.BlockSpec((_RBLK, _NS16), lambda i: (i, 0)),
            pl.BlockSpec((_BBLK, _DA), lambda i: (i, 0)),
            wspec((_NS16, _DA)),
            wspec((1, _DA)),
            wspec((1, _DA)),
            wspec((1, _DA)),
            wspec((1, _DA)),
            wspec((_DA, _D)),
            wspec((1, _D)),
            wspec((_DA, _D)),
            wspec((1, _D)),
        ],
        out_specs=[
            pl.BlockSpec((_BBLK, _D), lambda i: (i, 0)),
            pl.BlockSpec((_BBLK, _D), lambda i: (i, 0)),
        ],
        out_shape=[
            jax.ShapeDtypeStruct((_B, _D), jnp.float32),
            jax.ShapeDtypeStruct((_B, _D), jnp.float32),
        ],
    )(nf, s16, pdst, wtil, w_struct, bias_sum, ln_g, ln_b,
      wg_aug, b_gcn, wo_aug, b_out)


def _pad_lanes(row):
    return jnp.concatenate(
        [row.reshape(1, -1),
         jnp.zeros((1, _DA - row.reshape(1, -1).shape[1]), jnp.float32)],
        axis=1)


def kernel(src_neighbor_node_ids, src_neighbor_edge_ids, src_neighbor_times,
           dst_node_ids, node_raw_features, edge_raw_features,
           W_feat, b_feat, W_edge, b_edge, W_time, b_time, W_struct, b_struct,
           time_w, time_b, ln_g, ln_b, W_gcn, b_gcn, W_out, b_out):
    ids = src_neighbor_node_ids.astype(jnp.int32)
    eids = src_neighbor_edge_ids.astype(jnp.int32)
    dst = dst_node_ids.astype(jnp.int32)

    p_tab = _project_table(node_raw_features, W_feat, b_feat)

    pad = ((0, 0), (0, _NP - _N))
    ids2 = jnp.pad(ids, pad)
    eids2 = jnp.pad(eids, pad)
    times2 = jnp.pad(src_neighbor_times, pad)
    edge_col0 = edge_raw_features[:, 0]

    nf, e0, pdst = _sc_gather(
        p_tab, edge_col0, ids2.reshape(_R), eids2.reshape(_R), dst)

    bias_sum = _pad_lanes(b_edge + b_time + 2.0 * b_struct)
    tw_tb = jnp.stack([time_w, time_b], axis=0)

    s_wide = _pre(ids2, times2, e0.reshape(_B, _NP), dst.reshape(_B, 1),
                  tw_tb)
    s16 = s_wide.transpose(1, 2, 0).reshape(_R, _NS16)
    wtil = jnp.concatenate(
        [W_edge, W_struct, W_time, jnp.zeros((6, _D), jnp.float32)], axis=0)
    wtil = jnp.concatenate(
        [wtil, jnp.zeros((_NS16, _DA - _D), jnp.float32)], axis=1)
    zpad = jnp.zeros((_DA - _D, _D), jnp.float32)
    wg_aug = jnp.concatenate([W_gcn, zpad], axis=0)
    wo_aug = jnp.concatenate([W_out, zpad], axis=0)

    src_emb, dst_emb = _fuse(
        nf, s16, pdst, wtil, _pad_lanes(W_struct[0]), bias_sum,
        _pad_lanes(ln_g), _pad_lanes(ln_b),
        wg_aug, b_gcn.reshape(1, _D), wo_aug, b_out.reshape(1, _D))
    return (src_emb, dst_emb)


# speedup vs baseline: 1.3099x; 1.3099x over previous
import functools

import jax
import jax.numpy as jnp
from jax import lax
from jax.experimental import pallas as pl
from jax.experimental.pallas import tpu as pltpu
from jax.experimental.pallas import tpu_sc as plsc

_B = 4096
_N = 100
_NP = 104
_NNODES = 100000
_NEDGES = 1600000
_RAW = 128
_ERAW = 16
_TD = 8
_D = 64

_R = _B * _NP

_NC = 2
_NS = 16
_NW = _NC * _NS
_PW = _R // _NW
_CH = 832
_NCHUNK = _PW // _CH
_BD = _B // _NW

_BBLK = 128
_RBLK = _BBLK * _NP


def _proj_body(x_ref, w_ref, b_ref, o_ref):
    o_ref[...] = (
        jnp.dot(x_ref[...], w_ref[...], preferred_element_type=jnp.float32)
        + b_ref[...]
    )


def _project_table(node_raw, w_feat, b_feat):
    rb = 2000
    return pl.pallas_call(
        _proj_body,
        grid=(_NNODES // rb,),
        in_specs=[
            pl.BlockSpec((rb, _RAW), lambda i: (i, 0)),
            pl.BlockSpec((_RAW, _D), lambda i: (0, 0)),
            pl.BlockSpec((1, _D), lambda i: (0, 0)),
        ],
        out_specs=pl.BlockSpec((rb, _D), lambda i: (i, 0)),
        out_shape=jax.ShapeDtypeStruct((_NNODES, _D), jnp.float32),
    )(node_raw, w_feat, b_feat.reshape(1, _D))


def _sc_gather(p_tab, node_col0, edge_tab, ids_flat, eids_flat, dst):
    mesh = plsc.VectorSubcoreMesh(
        core_axis_name="c", subcore_axis_name="s",
        num_cores=_NC, num_subcores=_NS,
    )
    out_type = [
        jax.ShapeDtypeStruct((_R, _D), jnp.float32),
        jax.ShapeDtypeStruct((_R,), jnp.float32),
        jax.ShapeDtypeStruct((_R,), jnp.float32),
        jax.ShapeDtypeStruct((_B, _D), jnp.float32),
        jax.ShapeDtypeStruct((_B,), jnp.float32),
    ]
    chunk_bufs = lambda: [
        pltpu.VMEM((_CH,), jnp.int32),
        pltpu.VMEM((_CH,), jnp.int32),
        pltpu.VMEM((_CH, _D), jnp.float32),
        pltpu.VMEM((_CH,), jnp.float32),
        pltpu.VMEM((_CH,), jnp.float32),
        pltpu.SemaphoreType.DMA,
        pltpu.SemaphoreType.DMA,
        pltpu.SemaphoreType.DMA,
        pltpu.SemaphoreType.DMA,
    ]
    scratch = chunk_bufs() + chunk_bufs() + [
        pltpu.VMEM((_BD,), jnp.int32),
        pltpu.VMEM((_BD, _D), jnp.float32),
        pltpu.VMEM((_BD,), jnp.float32),
    ]

    @functools.partial(pl.kernel, mesh=mesh, out_type=out_type,
                       scratch_types=scratch,
                       compiler_params=pltpu.CompilerParams(
                           use_tc_tiling_on_sc=False,
                           needs_layout_passes=False))
    def k(p_h, c0_h, e0t_h, ids_h, eids_h, dst_h,
          nf_o, rsk_o, e0_o, pd_o, csk_o,
          idx_a, eidx_a, rows_a, e0_a, sk_a, sga, sea, ssa, swa,
          idx_b, eidx_b, rows_b, e0_b, sk_b, sgb, seb, ssb, swb,
          didx_v, drows_v, dsk_v):
        wid = lax.axis_index("s") * _NC + lax.axis_index("c")
        base = wid * _PW

        bufs_a = (idx_a, eidx_a, rows_a, e0_a, sk_a, sga, sea, ssa, swa)
        bufs_b = (idx_b, eidx_b, rows_b, e0_b, sk_b, sgb, seb, ssb, swb)

        def load_idx(bufs, off):
            idx, eidx = bufs[0], bufs[1]
            pltpu.sync_copy(ids_h.at[pl.ds(off, _CH)], idx)
            pltpu.sync_copy(eids_h.at[pl.ds(off, _CH)], eidx)

        def issue_gathers(bufs):
            idx, eidx, rows, e0_v, sk = bufs[0], bufs[1], bufs[2], bufs[3], bufs[4]
            sg, se, ss = bufs[5], bufs[6], bufs[7]
            return (pltpu.async_copy(p_h.at[idx], rows, sg),
                    pltpu.async_copy(e0t_h.at[eidx], e0_v, se),
                    pltpu.async_copy(c0_h.at[idx], sk, ss))

        def issue_writebacks(bufs, off):
            rows, e0_v, sk, sw = bufs[2], bufs[3], bufs[4], bufs[8]
            return (pltpu.async_copy(rows, nf_o.at[pl.ds(off, _CH)], sw),
                    pltpu.async_copy(e0_v, e0_o.at[pl.ds(off, _CH)], sw),
                    pltpu.async_copy(sk, rsk_o.at[pl.ds(off, _CH)], sw))

        def pair(ci, carry):
            off_a = base + (2 * ci) * _CH
            off_b = off_a + _CH
            load_idx(bufs_a, off_a)
            ga = issue_gathers(bufs_a)
            load_idx(bufs_b, off_b)
            gb = issue_gathers(bufs_b)
            for cp in ga:
                cp.wait()
            wa = issue_writebacks(bufs_a, off_a)
            for cp in gb:
                cp.wait()
            wb = issue_writebacks(bufs_b, off_b)
            for cp in wa + wb:
                cp.wait()
            return carry

        lax.fori_loop(0, _NCHUNK // 2, pair, 0)

        dbase = wid * _BD
        pltpu.sync_copy(dst_h.at[pl.ds(dbase, _BD)], didx_v)
        pltpu.async_copy(p_h.at[didx_v], drows_v, sga).wait()
        pltpu.sync_copy(drows_v, pd_o.at[pl.ds(dbase, _BD)])
        pltpu.async_copy(c0_h.at[didx_v], dsk_v, sea).wait()
        pltpu.sync_copy(dsk_v, csk_o.at[pl.ds(dbase, _BD)])

    return k(p_tab, node_col0, edge_tab, ids_flat, eids_flat, dst)


_NS16 = 16
_BP = 512


def _pre_body(ids_ref, t_ref, e0_ref, rsk_ref, dst_ref, csk_ref, twb_ref,
              o_ref):
    ids = ids_ref[...]
    t = t_ref[...]
    e0w = e0_ref[...]
    rskw = rsk_ref[...]
    dstc = dst_ref[...]
    cskc = csk_ref[...]

    valid = (ids > 0).astype(jnp.int32)
    vi = valid.sum(axis=-1, keepdims=True)
    n_l = lax.broadcasted_iota(jnp.int32, (_BP, _NP), 1)
    has_in = ((n_l >= 1) & (n_l <= vi - 1)).astype(jnp.float32)
    hprev = ((n_l - 1 >= 1) & (n_l - 1 <= vi - 1)).astype(jnp.float32)
    inv_deg = 1.0 - 0.5 * has_in
    coef = has_in * lax.rsqrt((1.0 + hprev) * (1.0 + has_in))
    co = (ids == dstc).astype(jnp.float32)
    ss = (rskw.astype(jnp.int32) == cskc.astype(jnp.int32))
    flags = co + ss.astype(jnp.float32)

    planes = [e0w, flags]
    for k in range(_TD):
        planes.append(jnp.cos(t * twb_ref[0, k] + twb_ref[1, k]))
    planes.append(inv_deg)
    planes.append(coef)
    z = jnp.zeros_like(e0w)
    planes.extend([z, z, z, z])
    o_ref[...] = jnp.stack(planes, axis=0)


def _pre(ids2, times2, e02, rsk2, dstc, cskc, tw_tb):
    return pl.pallas_call(
        _pre_body,
        grid=(_B // _BP,),
        in_specs=[
            pl.BlockSpec((_BP, _NP), lambda i: (i, 0)),
            pl.BlockSpec((_BP, _NP), lambda i: (i, 0)),
            pl.BlockSpec((_BP, _NP), lambda i: (i, 0)),
            pl.BlockSpec((_BP, _NP), lambda i: (i, 0)),
            pl.BlockSpec((_BP, 1), lambda i: (i, 0)),
            pl.BlockSpec((_BP, 1), lambda i: (i, 0)),
            pl.BlockSpec(memory_space=pltpu.SMEM),
        ],
        out_specs=pl.BlockSpec((_NS16, _BP, _NP), lambda i: (0, i, 0)),
        out_shape=jax.ShapeDtypeStruct((_NS16, _B, _NP), jnp.float32),
    )(ids2, times2, e02, rsk2, dstc, cskc, tw_tb)


def _fuse_body(nf_ref, s_ref, pd_ref, wtil_ref, bs_ref, lg_ref, lb_ref,
               wg_ref, bg_ref, wo_ref, bo_ref,
               src_ref, dstout_ref):
    x = nf_ref[...]
    s = s_ref[...]

    acc = (x + jnp.dot(s, wtil_ref[...], preferred_element_type=jnp.float32)
           + bs_ref[...])

    mu = jnp.mean(acc, axis=-1, keepdims=True)
    xc = acc - mu
    var = jnp.mean(xc * xc, axis=-1, keepdims=True)
    fused = xc * lax.rsqrt(var + 1e-5) * lg_ref[...] + lb_ref[...]

    inv_deg = s[:, 10:11]
    coef = s[:, 11:12]
    xprev = pltpu.roll(fused, 1, 0)
    agg = fused * inv_deg + coef * xprev

    gcn = jnp.maximum(
        jnp.dot(agg, wg_ref[...], preferred_element_type=jnp.float32)
        + bg_ref[...], 0.0)
    y = jnp.dot(gcn, wo_ref[...], preferred_element_type=jnp.float32)
    pooled = y.reshape(_BBLK, _NP, _D)[:, :_N, :].sum(axis=1) * (1.0 / _N)
    src_ref[...] = pooled + bo_ref[...]
    dstout_ref[...] = (
        jnp.dot(pd_ref[...], wo_ref[...], preferred_element_type=jnp.float32)
        + bo_ref[...])


def _fuse(nf, s16, pdst, wtil, bias_sum, ln_g, ln_b, w_gcn, b_gcn,
          w_out, b_out):
    grid = (_B // _BBLK,)
    wspec = lambda shape: pl.BlockSpec(shape, lambda i: (0,) * len(shape))
    return pl.pallas_call(
        _fuse_body,
        grid=grid,
        in_specs=[
            pl.BlockSpec((_RBLK, _D), lambda i: (i, 0)),
            pl.BlockSpec((_RBLK, _NS16), lambda i: (i, 0)),
            pl.BlockSpec((_BBLK, _D), lambda i: (i, 0)),
            wspec((_NS16, _D)),
            wspec((1, _D)),
            wspec((1, _D)),
            wspec((1, _D)),
            wspec((_D, _D)),
            wspec((1, _D)),
            wspec((_D, _D)),
            wspec((1, _D)),
        ],
        out_specs=[
            pl.BlockSpec((_BBLK, _D), lambda i: (i, 0)),
            pl.BlockSpec((_BBLK, _D), lambda i: (i, 0)),
        ],
        out_shape=[
            jax.ShapeDtypeStruct((_B, _D), jnp.float32),
            jax.ShapeDtypeStruct((_B, _D), jnp.float32),
        ],
    )(nf, s16, pdst, wtil, bias_sum, ln_g, ln_b, w_gcn, b_gcn, w_out, b_out)


def kernel(src_neighbor_node_ids, src_neighbor_edge_ids, src_neighbor_times,
           dst_node_ids, node_raw_features, edge_raw_features,
           W_feat, b_feat, W_edge, b_edge, W_time, b_time, W_struct, b_struct,
           time_w, time_b, ln_g, ln_b, W_gcn, b_gcn, W_out, b_out):
    ids = src_neighbor_node_ids.astype(jnp.int32)
    eids = src_neighbor_edge_ids.astype(jnp.int32)
    dst = dst_node_ids.astype(jnp.int32)

    p_tab = _project_table(node_raw_features, W_feat, b_feat)

    pad = ((0, 0), (0, _NP - _N))
    ids2 = jnp.pad(ids, pad)
    eids2 = jnp.pad(eids, pad)
    times2 = jnp.pad(src_neighbor_times, pad)
    node_col0 = node_raw_features[:, 0]
    edge_col0 = edge_raw_features[:, 0]

    nf, rsk, e0, pdst, csk = _sc_gather(
        p_tab, node_col0, edge_col0,
        ids2.reshape(_R), eids2.reshape(_R), dst)

    bias_sum = (b_feat + b_edge + b_time + 2.0 * b_struct).reshape(1, _D)
    tw_tb = jnp.stack([time_w, time_b], axis=0)

    s_wide = _pre(ids2, times2, e0.reshape(_B, _NP), rsk.reshape(_B, _NP),
                  dst.reshape(_B, 1), csk.reshape(_B, 1), tw_tb)
    s16 = s_wide.transpose(1, 2, 0).reshape(_R, _NS16)
    wtil = jnp.concatenate(
        [W_edge, W_struct, W_time, jnp.zeros((6, _D), jnp.float32)], axis=0)

    src_emb, dst_emb = _fuse(
        nf, s16, pdst, wtil, bias_sum,
        ln_g.reshape(1, _D), ln_b.reshape(1, _D),
        W_gcn, b_gcn.reshape(1, _D), W_out, b_out.reshape(1, _D))
    return (src_emb, dst_emb)


# speedup vs baseline: 1.3223x; 1.0095x over previous
import functools

import jax
import jax.numpy as jnp
from jax import lax
from jax.experimental import pallas as pl
from jax.experimental.pallas import tpu as pltpu
from jax.experimental.pallas import tpu_sc as plsc

_B = 4096
_N = 100
_NP = 104
_NNODES = 100000
_NEDGES = 1600000
_RAW = 128
_ERAW = 16
_TD = 8
_D = 64

_R = _B * _NP

_NC = 2
_NS = 16
_NW = _NC * _NS
_PW = _R // _NW
_CH = 832
_NCHUNK = _PW // _CH
_BD = _B // _NW

_BBLK = 128
_RBLK = _BBLK * _NP


def _proj_body(x_ref, w_ref, b_ref, o_ref):
    o_ref[...] = (
        jnp.dot(x_ref[...], w_ref[...], preferred_element_type=jnp.float32)
        + b_ref[...]
    )


def _project_table(node_raw, w_feat, b_feat):
    rb = 2000
    return pl.pallas_call(
        _proj_body,
        grid=(_NNODES // rb,),
        in_specs=[
            pl.BlockSpec((rb, _RAW), lambda i: (i, 0)),
            pl.BlockSpec((_RAW, _D), lambda i: (0, 0)),
            pl.BlockSpec((1, _D), lambda i: (0, 0)),
        ],
        out_specs=pl.BlockSpec((rb, _D), lambda i: (i, 0)),
        out_shape=jax.ShapeDtypeStruct((_NNODES, _D), jnp.float32),
    )(node_raw, w_feat, b_feat.reshape(1, _D))


_SCPARAMS = pltpu.CompilerParams(use_tc_tiling_on_sc=False,
                                 needs_layout_passes=False)


def _sc_mesh():
    return plsc.VectorSubcoreMesh(
        core_axis_name="c", subcore_axis_name="s",
        num_cores=_NC, num_subcores=_NS,
    )


_CHS = 1664


def _sc_scalars(node_col0, edge_col0, ids_flat, eids_flat, dst):
    out_type = [
        jax.ShapeDtypeStruct((_R,), jnp.float32),
        jax.ShapeDtypeStruct((_R,), jnp.float32),
        jax.ShapeDtypeStruct((_B,), jnp.float32),
    ]
    chunk_bufs = lambda: [
        pltpu.VMEM((_CHS,), jnp.int32),
        pltpu.VMEM((_CHS,), jnp.int32),
        pltpu.VMEM((_CHS,), jnp.float32),
        pltpu.VMEM((_CHS,), jnp.float32),
        pltpu.SemaphoreType.DMA,
        pltpu.SemaphoreType.DMA,
        pltpu.SemaphoreType.DMA,
    ]
    scratch = chunk_bufs() + chunk_bufs() + [
        pltpu.VMEM((_BD,), jnp.int32),
        pltpu.VMEM((_BD,), jnp.float32),
    ]

    @functools.partial(pl.kernel, mesh=_sc_mesh(), out_type=out_type,
                       scratch_types=scratch, compiler_params=_SCPARAMS)
    def k(c0_h, e0t_h, ids_h, eids_h, dst_h,
          rsk_o, e0_o, csk_o,
          idx_a, eidx_a, e0_a, sk_a, sea, ssa, swa,
          idx_b, eidx_b, e0_b, sk_b, seb, ssb, swb,
          didx_v, dsk_v):
        wid = lax.axis_index("s") * _NC + lax.axis_index("c")
        base = wid * _PW

        bufs_a = (idx_a, eidx_a, e0_a, sk_a, sea, ssa, swa)
        bufs_b = (idx_b, eidx_b, e0_b, sk_b, seb, ssb, swb)

        def load_idx(bufs, off):
            pltpu.sync_copy(ids_h.at[pl.ds(off, _CHS)], bufs[0])
            pltpu.sync_copy(eids_h.at[pl.ds(off, _CHS)], bufs[1])

        def issue_gathers(bufs):
            idx, eidx, e0_v, sk, se, ss = bufs[:6]
            return (pltpu.async_copy(e0t_h.at[eidx], e0_v, se),
                    pltpu.async_copy(c0_h.at[idx], sk, ss))

        def issue_writebacks(bufs, off):
            e0_v, sk, sw = bufs[2], bufs[3], bufs[6]
            return (pltpu.async_copy(e0_v, e0_o.at[pl.ds(off, _CHS)], sw),
                    pltpu.async_copy(sk, rsk_o.at[pl.ds(off, _CHS)], sw))

        def pair(ci, carry):
            off_a = base + (2 * ci) * _CHS
            off_b = off_a + _CHS
            load_idx(bufs_a, off_a)
            ga = issue_gathers(bufs_a)
            load_idx(bufs_b, off_b)
            gb = issue_gathers(bufs_b)
            for cp in ga:
                cp.wait()
            wa = issue_writebacks(bufs_a, off_a)
            for cp in gb:
                cp.wait()
            wb = issue_writebacks(bufs_b, off_b)
            for cp in wa + wb:
                cp.wait()
            return carry

        lax.fori_loop(0, _PW // _CHS // 2, pair, 0)

        dbase = wid * _BD
        pltpu.sync_copy(dst_h.at[pl.ds(dbase, _BD)], didx_v)
        pltpu.async_copy(c0_h.at[didx_v], dsk_v, sea).wait()
        pltpu.sync_copy(dsk_v, csk_o.at[pl.ds(dbase, _BD)])

    return k(node_col0, edge_col0, ids_flat, eids_flat, dst)


def _sc_rows(p_tab, ids_flat, dst):
    out_type = [
        jax.ShapeDtypeStruct((_R, _D), jnp.float32),
        jax.ShapeDtypeStruct((_B, _D), jnp.float32),
    ]
    chunk_bufs = lambda: [
        pltpu.VMEM((_CH,), jnp.int32),
        pltpu.VMEM((_CH, _D), jnp.float32),
        pltpu.SemaphoreType.DMA,
        pltpu.SemaphoreType.DMA,
    ]
    scratch = chunk_bufs() + chunk_bufs() + [
        pltpu.VMEM((_BD,), jnp.int32),
        pltpu.VMEM((_BD, _D), jnp.float32),
    ]

    @functools.partial(pl.kernel, mesh=_sc_mesh(), out_type=out_type,
                       scratch_types=scratch, compiler_params=_SCPARAMS)
    def k(p_h, ids_h, dst_h,
          nf_o, pd_o,
          idx_a, rows_a, sga, swa,
          idx_b, rows_b, sgb, swb,
          didx_v, drows_v):
        wid = lax.axis_index("s") * _NC + lax.axis_index("c")
        base = wid * _PW

        def pair(ci, carry):
            off_a = base + (2 * ci) * _CH
            off_b = off_a + _CH
            pltpu.sync_copy(ids_h.at[pl.ds(off_a, _CH)], idx_a)
            ga = pltpu.async_copy(p_h.at[idx_a], rows_a, sga)
            pltpu.sync_copy(ids_h.at[pl.ds(off_b, _CH)], idx_b)
            gb = pltpu.async_copy(p_h.at[idx_b], rows_b, sgb)
            ga.wait()
            wa = pltpu.async_copy(rows_a, nf_o.at[pl.ds(off_a, _CH)], swa)
            gb.wait()
            wb = pltpu.async_copy(rows_b, nf_o.at[pl.ds(off_b, _CH)], swb)
            wa.wait()
            wb.wait()
            return carry

        lax.fori_loop(0, _NCHUNK // 2, pair, 0)

        dbase = wid * _BD
        pltpu.sync_copy(dst_h.at[pl.ds(dbase, _BD)], didx_v)
        pltpu.async_copy(p_h.at[didx_v], drows_v, sga).wait()
        pltpu.sync_copy(drows_v, pd_o.at[pl.ds(dbase, _BD)])

    return k(p_tab, ids_flat, dst)


_NS16 = 16
_BP = 512


def _pre_body(ids_ref, t_ref, e0_ref, rsk_ref, dst_ref, csk_ref, twb_ref,
              o_ref):
    ids = ids_ref[...]
    t = t_ref[...]
    e0w = e0_ref[...]
    rskw = rsk_ref[...]
    dstc = dst_ref[...]
    cskc = csk_ref[...]

    valid = (ids > 0).astype(jnp.int32)
    vi = valid.sum(axis=-1, keepdims=True)
    n_l = lax.broadcasted_iota(jnp.int32, (_BP, _NP), 1)
    has_in = ((n_l >= 1) & (n_l <= vi - 1)).astype(jnp.float32)
    hprev = ((n_l - 1 >= 1) & (n_l - 1 <= vi - 1)).astype(jnp.float32)
    inv_deg = 1.0 - 0.5 * has_in
    coef = has_in * lax.rsqrt((1.0 + hprev) * (1.0 + has_in))
    co = (ids == dstc).astype(jnp.float32)
    ss = (rskw.astype(jnp.int32) == cskc.astype(jnp.int32))
    flags = co + ss.astype(jnp.float32)

    planes = [e0w, flags]
    for k in range(_TD):
        planes.append(jnp.cos(t * twb_ref[0, k] + twb_ref[1, k]))
    planes.append(inv_deg)
    planes.append(coef)
    z = jnp.zeros_like(e0w)
    planes.extend([z, z, z, z])
    o_ref[...] = jnp.stack(planes, axis=0)


def _pre(ids2, times2, e02, rsk2, dstc, cskc, tw_tb):
    return pl.pallas_call(
        _pre_body,
        grid=(_B // _BP,),
        in_specs=[
            pl.BlockSpec((_BP, _NP), lambda i: (i, 0)),
            pl.BlockSpec((_BP, _NP), lambda i: (i, 0)),
            pl.BlockSpec((_BP, _NP), lambda i: (i, 0)),
            pl.BlockSpec((_BP, _NP), lambda i: (i, 0)),
            pl.BlockSpec((_BP, 1), lambda i: (i, 0)),
            pl.BlockSpec((_BP, 1), lambda i: (i, 0)),
            pl.BlockSpec(memory_space=pltpu.SMEM),
        ],
        out_specs=pl.BlockSpec((_NS16, _BP, _NP), lambda i: (0, i, 0)),
        out_shape=jax.ShapeDtypeStruct((_NS16, _B, _NP), jnp.float32),
    )(ids2, times2, e02, rsk2, dstc, cskc, tw_tb)


def _fuse_body(nf_ref, s_ref, pd_ref, wtil_ref, bs_ref, lg_ref, lb_ref,
               wg_ref, bg_ref, wo_ref, bo_ref,
               src_ref, dstout_ref):
    x = nf_ref[...]
    s = s_ref[...]

    acc = (x + jnp.dot(s, wtil_ref[...], preferred_element_type=jnp.float32)
           + bs_ref[...])

    mu = jnp.mean(acc, axis=-1, keepdims=True)
    xc = acc - mu
    var = jnp.mean(xc * xc, axis=-1, keepdims=True)
    fused = xc * lax.rsqrt(var + 1e-5) * lg_ref[...] + lb_ref[...]

    inv_deg = s[:, 10:11]
    coef = s[:, 11:12]
    xprev = pltpu.roll(fused, 1, 0)
    agg = fused * inv_deg + coef * xprev

    gcn = jnp.maximum(
        jnp.dot(agg, wg_ref[...], preferred_element_type=jnp.float32)
        + bg_ref[...], 0.0)
    y = jnp.dot(gcn, wo_ref[...], preferred_element_type=jnp.float32)
    pooled = y.reshape(_BBLK, _NP, _D)[:, :_N, :].sum(axis=1) * (1.0 / _N)
    src_ref[...] = pooled + bo_ref[...]
    dstout_ref[...] = (
        jnp.dot(pd_ref[...], wo_ref[...], preferred_element_type=jnp.float32)
        + bo_ref[...])


def _fuse(nf, s16, pdst, wtil, bias_sum, ln_g, ln_b, w_gcn, b_gcn,
          w_out, b_out):
    grid = (_B // _BBLK,)
    wspec = lambda shape: pl.BlockSpec(shape, lambda i: (0,) * len(shape))
    return pl.pallas_call(
        _fuse_body,
        grid=grid,
        in_specs=[
            pl.BlockSpec((_RBLK, _D), lambda i: (i, 0)),
            pl.BlockSpec((_RBLK, _NS16), lambda i: (i, 0)),
            pl.BlockSpec((_BBLK, _D), lambda i: (i, 0)),
            wspec((_NS16, _D)),
            wspec((1, _D)),
            wspec((1, _D)),
            wspec((1, _D)),
            wspec((_D, _D)),
            wspec((1, _D)),
            wspec((_D, _D)),
            wspec((1, _D)),
        ],
        out_specs=[
            pl.BlockSpec((_BBLK, _D), lambda i: (i, 0)),
            pl.BlockSpec((_BBLK, _D), lambda i: (i, 0)),
        ],
        out_shape=[
            jax.ShapeDtypeStruct((_B, _D), jnp.float32),
            jax.ShapeDtypeStruct((_B, _D), jnp.float32),
        ],
    )(nf, s16, pdst, wtil, bias_sum, ln_g, ln_b, w_gcn, b_gcn, w_out, b_out)


def kernel(src_neighbor_node_ids, src_neighbor_edge_ids, src_neighbor_times,
           dst_node_ids, node_raw_features, edge_raw_features,
           W_feat, b_feat, W_edge, b_edge, W_time, b_time, W_struct, b_struct,
           time_w, time_b, ln_g, ln_b, W_gcn, b_gcn, W_out, b_out):
    ids = src_neighbor_node_ids.astype(jnp.int32)
    eids = src_neighbor_edge_ids.astype(jnp.int32)
    dst = dst_node_ids.astype(jnp.int32)

    p_tab = _project_table(node_raw_features, W_feat, b_feat)

    pad = ((0, 0), (0, _NP - _N))
    ids2 = jnp.pad(ids, pad)
    eids2 = jnp.pad(eids, pad)
    times2 = jnp.pad(src_neighbor_times, pad)
    node_col0 = node_raw_features[:, 0]
    edge_col0 = edge_raw_features[:, 0]

    rsk, e0, csk = _sc_scalars(
        node_col0, edge_col0, ids2.reshape(_R), eids2.reshape(_R), dst)
    nf, pdst = _sc_rows(p_tab, ids2.reshape(_R), dst)

    bias_sum = (b_feat + b_edge + b_time + 2.0 * b_struct).reshape(1, _D)
    tw_tb = jnp.stack([time_w, time_b], axis=0)

    s_wide = _pre(ids2, times2, e0.reshape(_B, _NP), rsk.reshape(_B, _NP),
                  dst.reshape(_B, 1), csk.reshape(_B, 1), tw_tb)
    s16 = s_wide.transpose(1, 2, 0).reshape(_R, _NS16)
    wtil = jnp.concatenate(
        [W_edge, W_struct, W_time, jnp.zeros((6, _D), jnp.float32)], axis=0)

    src_emb, dst_emb = _fuse(
        nf, s16, pdst, wtil, bias_sum,
        ln_g.reshape(1, _D), ln_b.reshape(1, _D),
        W_gcn, b_gcn.reshape(1, _D), W_out, b_out.reshape(1, _D))
    return (src_emb, dst_emb)
